# baseline (speedup 1.0000x reference)
import jax
import jax.numpy as jnp
from jax import lax
from jax.experimental import pallas as pl
from jax.experimental.pallas import tpu as pltpu


def _gating_kernel(x_ref, wg_ref, bg_ref, aux_ref):
    BT = x_ref.shape[0]
    E = wg_ref.shape[1]
    logits = jnp.dot(x_ref[...], wg_ref[...], preferred_element_type=jnp.float32)
    logits = logits + bg_ref[...]
    m = jnp.max(logits, axis=-1, keepdims=True)
    ex = jnp.exp(logits - m)
    scores = ex / jnp.sum(ex, axis=-1, keepdims=True)
    col = lax.broadcasted_iota(jnp.int32, scores.shape, 1)
    rank = jnp.zeros(scores.shape, jnp.float32)
    for ep in range(E):
        c = scores[:, ep:ep + 1]
        rank = rank + (c > scores).astype(jnp.float32)
        rank = rank + ((c == scores) & (ep < col)).astype(jnp.float32)
    sel1 = (rank == 0.0).astype(jnp.float32)
    sel2 = (rank == 1.0).astype(jnp.float32)
    colf = col.astype(jnp.float32)
    e1 = jnp.sum(sel1 * colf, axis=-1, keepdims=True)
    e2 = jnp.sum(sel2 * colf, axis=-1, keepdims=True)
    v1 = jnp.sum(sel1 * scores, axis=-1, keepdims=True)
    v2 = jnp.sum(sel2 * scores, axis=-1, keepdims=True)
    den = jnp.maximum(v1 + v2, 1e-12)
    v1n = v1 / den
    v2n = v2 / den
    s = v1n + v2n
    rows = lax.broadcasted_iota(jnp.int32, (BT, 1), 0)
    rbase = ((rows // E) * E).astype(jnp.float32)
    aux_ref[:, 0:1] = rbase + e1
    aux_ref[:, 1:2] = rbase + e2
    aux_ref[:, 2:3] = v1n
    aux_ref[:, 3:4] = v2n
    aux_ref[:, 4:5] = s
    aux_ref[:, 5:8] = jnp.zeros((BT, 3), jnp.float32)


def _expert_kernel(x_ref, aux_ref, w1_ref, b1_ref, w2_ref, b2_ref, out_ref):
    t = pl.program_id(0)
    BT, D = x_ref.shape
    B, T, O = out_ref.shape
    S = 2 * B

    jb = lax.broadcasted_iota(jnp.int32, (S, BT), 0)
    jr = lax.broadcasted_iota(jnp.int32, (S, BT), 1)
    ssum = (jr == (jb // 2) * T + t).astype(jnp.float32)
    g = jnp.dot(ssum, aux_ref[...], preferred_element_type=jnp.float32,
                precision=lax.Precision.HIGHEST)
    odd = lax.broadcasted_iota(jnp.int32, (S, 1), 0) % 2
    u = jnp.where(odd == 0, g[:, 0:1], g[:, 1:2])
    w = jnp.where(odd == 0, g[:, 2:3], g[:, 3:4])

    p = (jr == u.astype(jnp.int32)).astype(jnp.float32)
    xsel = jnp.dot(p, x_ref[...], preferred_element_type=jnp.float32)

    h = jnp.dot(xsel, w1_ref[0], preferred_element_type=jnp.float32)
    h = jnp.maximum(h + b1_ref[0], 0.0)
    hw = h * w

    cb = lax.broadcasted_iota(jnp.int32, (B, S), 0)
    cj = lax.broadcasted_iota(jnp.int32, (B, S), 1)
    comb = (cj // 2 == cb).astype(jnp.float32)
    mixed = jnp.dot(comb, hw, preferred_element_type=jnp.float32)
    s128 = jnp.dot(comb, w, preferred_element_type=jnp.float32,
                   precision=lax.Precision.HIGHEST)

    out = jnp.dot(mixed, w2_ref[0], preferred_element_type=jnp.float32)
    res = out + s128 * b2_ref[0]
    out_ref[:, pl.ds(t, 1), :] = res[:, None, :]


def kernel(x, num_experts_chosen, Wg, bg, W1, b1, W2, b2):
    del num_experts_chosen
    B, T, D = x.shape
    E, _, H = W1.shape
    O = W2.shape[2]
    BT = B * T
    x2 = x.reshape(BT, D)
    bg2 = bg.reshape(1, E)
    b1_3 = b1.reshape(E, 1, H)
    b2_3 = b2.reshape(E, 1, O)

    aux = pl.pallas_call(
        _gating_kernel,
        in_specs=[
            pl.BlockSpec((BT, D), lambda: (0, 0)),
            pl.BlockSpec((D, E), lambda: (0, 0)),
            pl.BlockSpec((1, E), lambda: (0, 0)),
        ],
        out_specs=pl.BlockSpec((BT, 8), lambda: (0, 0)),
        out_shape=jax.ShapeDtypeStruct((BT, 8), jnp.float32),
    )(x2, Wg, bg2)

    out = pl.pallas_call(
        _expert_kernel,
        grid=(E,),
        in_specs=[
            pl.BlockSpec((BT, D), lambda t: (0, 0)),
            pl.BlockSpec((BT, 8), lambda t: (0, 0)),
            pl.BlockSpec((1, D, H), lambda t: (t, 0, 0)),
            pl.BlockSpec((1, 1, H), lambda t: (t, 0, 0)),
            pl.BlockSpec((1, H, O), lambda t: (t, 0, 0)),
            pl.BlockSpec((1, 1, O), lambda t: (t, 0, 0)),
        ],
        out_specs=pl.BlockSpec((B, T, O), lambda t: (0, 0, 0)),
        out_shape=jax.ShapeDtypeStruct((B, T, O), jnp.float32),
    )(x2, aux, W1, b1_3, W2, b2_3)
    return out


# speedup vs baseline: 1.1412x; 1.1412x over previous
import jax
import jax.numpy as jnp
from jax import lax
from jax.experimental import pallas as pl


def _gating_kernel(x_ref, wg_ref, bg_ref, wcomb_ref):
    BT = x_ref.shape[0]
    E = wg_ref.shape[1]
    B = wcomb_ref.shape[1]
    T = BT // B
    logits = jnp.dot(x_ref[...], wg_ref[...], preferred_element_type=jnp.float32)
    logits = logits + bg_ref[...]
    m = jnp.max(logits, axis=-1, keepdims=True)
    ex = jnp.exp(logits - m)
    scores = ex / jnp.sum(ex, axis=-1, keepdims=True)

    col = lax.broadcasted_iota(jnp.int32, scores.shape, 1)
    m1 = jnp.max(scores, axis=-1, keepdims=True)
    e1 = jnp.min(jnp.where(scores == m1, col, E), axis=-1, keepdims=True)
    rest = jnp.where(col == e1, -jnp.inf, scores)
    m2 = jnp.max(rest, axis=-1, keepdims=True)
    e2 = jnp.min(jnp.where(rest == m2, col, E), axis=-1, keepdims=True)
    sel = (col == e1) | (col == e2)
    gated = jnp.where(sel, scores, 0.0)
    den = jnp.maximum(jnp.sum(gated, axis=-1, keepdims=True), 1e-12)
    gated = gated / den

    iota_b = lax.broadcasted_iota(jnp.int32, (B, BT), 0)
    iota_r = lax.broadcasted_iota(jnp.int32, (B, BT), 1)
    blockdiag = (iota_r // T == iota_b).astype(jnp.float32)
    it_e = lax.broadcasted_iota(jnp.int32, (E, BT), 0)
    it_r = lax.broadcasted_iota(jnp.int32, (E, BT), 1)
    tile = (it_r % T == it_e).astype(jnp.float32)
    for t in range(T):
        rsel = (iota_r == iota_b * T + t).astype(jnp.float32)
        gt = jnp.dot(rsel, gated, preferred_element_type=jnp.float32,
                     precision=lax.Precision.HIGHEST)
        w = jnp.dot(gt, tile, preferred_element_type=jnp.float32,
                    precision=lax.Precision.HIGHEST) * blockdiag
        wcomb_ref[t] = w.astype(jnp.bfloat16)


def _expert_kernel(x_ref, wc_ref, w1_ref, b1_ref, w2_ref, b2_ref, out_ref):
    t = pl.program_id(0)
    h = jnp.dot(x_ref[...], w1_ref[0], preferred_element_type=jnp.float32)
    h = jnp.maximum(h + b1_ref[0], 0.0)
    mixed = jnp.dot(wc_ref[0], h, preferred_element_type=jnp.float32)
    out = jnp.dot(mixed, w2_ref[0], preferred_element_type=jnp.float32)
    res = out + b2_ref[0]
    out_ref[:, pl.ds(t, 1), :] = res[:, None, :]


def kernel(x, num_experts_chosen, Wg, bg, W1, b1, W2, b2):
    del num_experts_chosen
    B, T, D = x.shape
    E, _, H = W1.shape
    O = W2.shape[2]
    BT = B * T
    x2 = x.reshape(BT, D)
    bg2 = bg.reshape(1, E)
    b1_3 = b1.reshape(E, 1, H)
    b2_3 = b2.reshape(E, 1, O)

    wcomb = pl.pallas_call(
        _gating_kernel,
        in_specs=[
            pl.BlockSpec((BT, D), lambda: (0, 0)),
            pl.BlockSpec((D, E), lambda: (0, 0)),
            pl.BlockSpec((1, E), lambda: (0, 0)),
        ],
        out_specs=pl.BlockSpec((T, B, BT), lambda: (0, 0, 0)),
        out_shape=jax.ShapeDtypeStruct((T, B, BT), jnp.bfloat16),
    )(x2, Wg, bg2)

    out = pl.pallas_call(
        _expert_kernel,
        grid=(E,),
        in_specs=[
            pl.BlockSpec((BT, D), lambda t: (0, 0)),
            pl.BlockSpec((1, B, BT), lambda t: (t, 0, 0)),
            pl.BlockSpec((1, D, H), lambda t: (t, 0, 0)),
            pl.BlockSpec((1, 1, H), lambda t: (t, 0, 0)),
            pl.BlockSpec((1, H, O), lambda t: (t, 0, 0)),
            pl.BlockSpec((1, 1, O), lambda t: (t, 0, 0)),
        ],
        out_specs=pl.BlockSpec((B, T, O), lambda t: (0, 0, 0)),
        out_shape=jax.ShapeDtypeStruct((B, T, O), jnp.float32),
    )(x2, wcomb, W1, b1_3, W2, b2_3)
    return out


# speedup vs baseline: 1.2794x; 1.1211x over previous
import jax
import jax.numpy as jnp
from jax import lax
from jax.experimental import pallas as pl


def _gating_kernel(x_ref, wg_ref, bg_ref, wcomb_ref):
    BT = x_ref.shape[0]
    E = wg_ref.shape[1]
    B = wcomb_ref.shape[1]
    T = BT // B
    logits = jnp.dot(x_ref[...], wg_ref[...], preferred_element_type=jnp.float32)
    logits = logits + bg_ref[...]
    m = jnp.max(logits, axis=-1, keepdims=True)
    ex = jnp.exp(logits - m)
    scores = ex / jnp.sum(ex, axis=-1, keepdims=True)

    col = lax.broadcasted_iota(jnp.int32, scores.shape, 1)
    m1 = jnp.max(scores, axis=-1, keepdims=True)
    e1 = jnp.min(jnp.where(scores == m1, col, E), axis=-1, keepdims=True)
    rest = jnp.where(col == e1, -jnp.inf, scores)
    m2 = jnp.max(rest, axis=-1, keepdims=True)
    e2 = jnp.min(jnp.where(rest == m2, col, E), axis=-1, keepdims=True)
    sel = (col == e1) | (col == e2)
    gated = jnp.where(sel, scores, 0.0)
    den = jnp.maximum(jnp.sum(gated, axis=-1, keepdims=True), 1e-12)
    gated = gated / den

    iota_b = lax.broadcasted_iota(jnp.int32, (B, BT), 0)
    iota_r = lax.broadcasted_iota(jnp.int32, (B, BT), 1)
    blockdiag = (iota_r // T == iota_b).astype(jnp.float32)
    it_e = lax.broadcasted_iota(jnp.int32, (E, BT), 0)
    it_r = lax.broadcasted_iota(jnp.int32, (E, BT), 1)
    tile = (it_r % T == it_e).astype(jnp.float32)
    for t in range(T):
        rsel = (iota_r == iota_b * T + t).astype(jnp.float32)
        gt = jnp.dot(rsel, gated, preferred_element_type=jnp.float32)
        w = jnp.dot(gt, tile, preferred_element_type=jnp.float32) * blockdiag
        wcomb_ref[t] = w.astype(jnp.bfloat16)


def _expert_kernel(x_ref, wc_ref, w1_ref, b1_ref, w2_ref, b2_ref, out_ref):
    t = pl.program_id(0)
    h = jnp.dot(x_ref[...], w1_ref[0], preferred_element_type=jnp.float32)
    h = jnp.maximum(h + b1_ref[0], 0.0)
    mixed = jnp.dot(wc_ref[0], h, preferred_element_type=jnp.float32)
    out = jnp.dot(mixed, w2_ref[0], preferred_element_type=jnp.float32)
    res = out + b2_ref[0]
    out_ref[:, pl.ds(t, 1), :] = res[:, None, :]


def kernel(x, num_experts_chosen, Wg, bg, W1, b1, W2, b2):
    del num_experts_chosen
    B, T, D = x.shape
    E, _, H = W1.shape
    O = W2.shape[2]
    BT = B * T
    x2 = x.reshape(BT, D)
    bg2 = bg.reshape(1, E)
    b1_3 = b1.reshape(E, 1, H)
    b2_3 = b2.reshape(E, 1, O)

    wcomb = pl.pallas_call(
        _gating_kernel,
        in_specs=[
            pl.BlockSpec((BT, D), lambda: (0, 0)),
            pl.BlockSpec((D, E), lambda: (0, 0)),
            pl.BlockSpec((1, E), lambda: (0, 0)),
        ],
        out_specs=pl.BlockSpec((T, B, BT), lambda: (0, 0, 0)),
        out_shape=jax.ShapeDtypeStruct((T, B, BT), jnp.bfloat16),
    )(x2, Wg, bg2)

    out = pl.pallas_call(
        _expert_kernel,
        grid=(E,),
        in_specs=[
            pl.BlockSpec((BT, D), lambda t: (0, 0)),
            pl.BlockSpec((1, B, BT), lambda t: (t, 0, 0)),
            pl.BlockSpec((1, D, H), lambda t: (t, 0, 0)),
            pl.BlockSpec((1, 1, H), lambda t: (t, 0, 0)),
            pl.BlockSpec((1, H, O), lambda t: (t, 0, 0)),
            pl.BlockSpec((1, 1, O), lambda t: (t, 0, 0)),
        ],
        out_specs=pl.BlockSpec((B, T, O), lambda t: (0, 0, 0)),
        out_shape=jax.ShapeDtypeStruct((B, T, O), jnp.float32),
    )(x2, wcomb, W1, b1_3, W2, b2_3)
    return out


# speedup vs baseline: 1.5568x; 1.2168x over previous
import jax
import jax.numpy as jnp
from jax import lax
from jax.experimental import pallas as pl
from jax.experimental.pallas import tpu as pltpu


def _moe_kernel(x_hbm, wg_ref, bg_ref, w1_hbm, b1_ref, w2_hbm, b2_ref,
                out_ref, xv, w1buf, w2buf, sem_x, sem_w1, sem_w2):
    BT, D = xv.shape
    E = wg_ref.shape[1]
    B, T, O = out_ref.shape

    cx = pltpu.make_async_copy(x_hbm, xv, sem_x)
    cx.start()
    pltpu.make_async_copy(w1_hbm.at[0], w1buf.at[0], sem_w1.at[0]).start()
    pltpu.make_async_copy(w2_hbm.at[0], w2buf.at[0], sem_w2.at[0]).start()
    cx.wait()
    X = xv[...]

    logits = jnp.dot(X, wg_ref[...], preferred_element_type=jnp.float32)
    logits = logits + bg_ref[...]
    m = jnp.max(logits, axis=-1, keepdims=True)
    ex = jnp.exp(logits - m)
    scores = ex / jnp.sum(ex, axis=-1, keepdims=True)
    col = lax.broadcasted_iota(jnp.int32, scores.shape, 1)
    m1 = jnp.max(scores, axis=-1, keepdims=True)
    e1 = jnp.min(jnp.where(scores == m1, col, E), axis=-1, keepdims=True)
    rest = jnp.where(col == e1, -jnp.inf, scores)
    m2 = jnp.max(rest, axis=-1, keepdims=True)
    e2 = jnp.min(jnp.where(rest == m2, col, E), axis=-1, keepdims=True)
    sel = (col == e1) | (col == e2)
    gated = jnp.where(sel, scores, 0.0)
    den = jnp.maximum(jnp.sum(gated, axis=-1, keepdims=True), 1e-12)
    gated = gated / den

    iota_b = lax.broadcasted_iota(jnp.int32, (B, BT), 0)
    iota_r = lax.broadcasted_iota(jnp.int32, (B, BT), 1)
    blockdiag = (iota_r // T == iota_b).astype(jnp.float32)
    it_e = lax.broadcasted_iota(jnp.int32, (E, BT), 0)
    it_r = lax.broadcasted_iota(jnp.int32, (E, BT), 1)
    tile = (it_r % T == it_e).astype(jnp.float32)

    for t in range(T):
        slot = t % 2
        nxt = (t + 1) % 2
        if t + 1 < T:
            pltpu.make_async_copy(w1_hbm.at[t + 1], w1buf.at[nxt],
                                  sem_w1.at[nxt]).start()
            pltpu.make_async_copy(w2_hbm.at[t + 1], w2buf.at[nxt],
                                  sem_w2.at[nxt]).start()
        pltpu.make_async_copy(w1_hbm.at[t], w1buf.at[slot],
                              sem_w1.at[slot]).wait()
        pltpu.make_async_copy(w2_hbm.at[t], w2buf.at[slot],
                              sem_w2.at[slot]).wait()

        rsel = (iota_r == iota_b * T + t).astype(jnp.float32)
        gt = jnp.dot(rsel, gated, preferred_element_type=jnp.float32)
        wcomb = jnp.dot(gt, tile, preferred_element_type=jnp.float32)
        wcomb = wcomb * blockdiag

        h = jnp.dot(X, w1buf[slot], preferred_element_type=jnp.float32)
        h = jnp.maximum(h + b1_ref[t:t + 1, :], 0.0)
        mixed = jnp.dot(wcomb, h, preferred_element_type=jnp.float32)
        out = jnp.dot(mixed, w2buf[slot], preferred_element_type=jnp.float32)
        out_ref[:, t, :] = out + b2_ref[t:t + 1, :]


def kernel(x, num_experts_chosen, Wg, bg, W1, b1, W2, b2):
    del num_experts_chosen
    B, T, D = x.shape
    E, _, H = W1.shape
    O = W2.shape[2]
    BT = B * T
    x2 = x.reshape(BT, D)
    bg2 = bg.reshape(1, E)

    hbm = pl.BlockSpec(memory_space=pltpu.MemorySpace.HBM)
    out = pl.pallas_call(
        _moe_kernel,
        in_specs=[
            hbm,
            pl.BlockSpec((D, E), lambda: (0, 0)),
            pl.BlockSpec((1, E), lambda: (0, 0)),
            hbm,
            pl.BlockSpec((E, H), lambda: (0, 0)),
            hbm,
            pl.BlockSpec((E, O), lambda: (0, 0)),
        ],
        out_specs=pl.BlockSpec((B, T, O), lambda: (0, 0, 0)),
        out_shape=jax.ShapeDtypeStruct((B, T, O), jnp.float32),
        scratch_shapes=[
            pltpu.VMEM((BT, D), jnp.float32),
            pltpu.VMEM((2, D, H), jnp.float32),
            pltpu.VMEM((2, H, O), jnp.float32),
            pltpu.SemaphoreType.DMA,
            pltpu.SemaphoreType.DMA((2,)),
            pltpu.SemaphoreType.DMA((2,)),
        ],
    )(x2, Wg, bg2, W1, b1, W2, b2)
    return out
